# baseline (speedup 1.0000x reference)
import jax
import jax.numpy as jnp
from jax.experimental import pallas as pl

DIM = 256
NUM_HEADS = 8
NUM_POINTS = 16
HEAD_DIM = DIM // NUM_HEADS

_QPTS = 128
_TN = 128


def _gelu(x):
    return 0.5 * x * (1.0 + jax.lax.erf(x * 0.7071067811865476))


def _ln(x, g, b):
    m = jnp.mean(x, axis=-1, keepdims=True)
    xc = x - m
    v = jnp.mean(xc * xc, axis=-1, keepdims=True)
    return xc * jax.lax.rsqrt(v + 1e-5) * g + b


def _dot(a, b):
    return jnp.dot(a, b, preferred_element_type=jnp.float32)


def _dot3(a, w):
    return (a[:, 0:1] * w[0:1, :] + a[:, 1:2] * w[1:2, :]
            + a[:, 2:3] * w[2:3, :])


def _pre_kernel(xt_ref, xyz_ref, ow1a_ref, ow1b_ref, ob1_ref, olng_ref,
                olnb_ref, ow2_ref, ob2_ref, ow3_ref, ob3_ref, scale_ref,
                wq_ref, wk_ref, wv_ref,
                off_ref, q_ref, k_ref, v_ref):
    xt = xt_ref[0]
    xyz = xyz_ref[0]
    h = _dot(xt, ow1a_ref[...]) + _dot3(xyz, ow1b_ref[...]) + ob1_ref[...]
    h = _gelu(_ln(h, olng_ref[...], olnb_ref[...]))
    h = _gelu(_dot(h, ow2_ref[...]) + ob2_ref[...])
    off = _dot(h, ow3_ref[...]) + ob3_ref[...]
    off = jnp.tanh(off) * jnp.abs(scale_ref[0, 0])
    off_ref[0] = off
    q_ref[0] = _dot(xt, wq_ref[...])
    k_ref[0] = _dot(xt, wk_ref[...])
    v_ref[0] = _dot(xt, wv_ref[...])


def _knn_kernel(qxyz_ref, off_ref, xyzt_ref, kvf_ref,
                rpw1_ref, rpb1_ref, rplng_ref, rplnb_ref, rpw2_ref, rpb2_ref,
                out_ref, rb_ref):
    off = off_ref[0]
    qxyz = jnp.broadcast_to(qxyz_ref[0][:, None, :],
                            (_QPTS, NUM_POINTS, 3)).reshape(_QPTS * NUM_POINTS, 3)
    qp = qxyz + off
    kx = xyzt_ref[0, 0:1, :]
    ky = xyzt_ref[0, 1:2, :]
    kz = xyzt_ref[0, 2:3, :]
    k3 = xyzt_ref[0]
    qhi = qp.astype(jnp.bfloat16).astype(jnp.float32)
    qlo = qp - qhi
    khi = k3.astype(jnp.bfloat16).astype(jnp.float32)
    klo = k3 - khi
    qaug = jnp.concatenate([-2.0 * qhi, -2.0 * qhi, -2.0 * qlo], axis=1)
    kaug = jnp.concatenate([khi, klo, khi], axis=0)
    qn = jnp.sum(qp * qp, axis=1, keepdims=True)
    kn = jnp.sum(k3 * k3, axis=0, keepdims=True)
    d2 = jnp.maximum(_dot(qaug, kaug) + qn + kn, 0.0)
    big = jnp.float32(3.0e38)
    nk = d2.shape[1]
    lw = nk // 8
    r1 = d2[:, 0:lw]
    r2 = jnp.full_like(r1, big)
    r3 = jnp.full_like(r1, big)
    for s_ in range(1, 8):
        xsl = d2[:, s_ * lw:(s_ + 1) * lw]
        t1 = jnp.minimum(r1, xsl)
        hi = jnp.maximum(r1, xsl)
        t2 = jnp.minimum(r2, hi)
        hi2 = jnp.maximum(r2, hi)
        t3 = jnp.minimum(r3, hi2)
        r1, r2, r3 = t1, t2, t3
    cand = jnp.concatenate([r1, r2, r3], axis=1)
    d1 = jnp.min(cand, axis=1, keepdims=True)
    m2 = jnp.min(jnp.where(cand > d1, cand, big), axis=1, keepdims=True)
    m3 = jnp.min(jnp.where(cand > m2, cand, big), axis=1, keepdims=True)
    dr0 = 1.0 / (d1 + 1e-8)
    dr1 = 1.0 / (m2 + 1e-8)
    dr2 = 1.0 / (m3 + 1e-8)
    rs = 1.0 / (dr0 + dr1 + dr2)
    w = jnp.where(d2 <= m3, (1.0 / (d2 + 1e-8)) * rs, 0.0)
    out_ref[0] = _dot(w, kvf_ref[0])

    rb = _gelu(_ln(_dot3(-off, rpw1_ref[...]) + rpb1_ref[...],
                   rplng_ref[...], rplnb_ref[...]))
    rb_ref[0] = (jnp.sum(rb * rpw2_ref[...], axis=1, keepdims=True)
                 + rpb2_ref[0, 0])


def _attn_kernel(xt_ref, q_ref, skv_ref, rb_ref,
                 wo_ref, n1g_ref, n1b_ref, fw1_ref, fb1_ref, fw2_ref, fb2_ref,
                 n2g_ref, n2b_ref, out_ref):
    H, P, D = NUM_HEADS, NUM_POINTS, HEAD_DIM
    xt = xt_ref[0]
    q4 = q_ref[0]
    skv4 = skv_ref[0]
    rb4 = rb_ref[0]
    scale = D ** -0.5
    wo = wo_ref[...]
    s3 = skv4.reshape(H * _TN, P, 2 * D)
    sk3 = s3[:, :, :D]
    sv3 = s3[:, :, D:]
    q3 = q4.reshape(H * _TN, 1, D)
    rb3 = rb4.reshape(H * _TN, P, 1)
    attn = jnp.sum(q3 * sk3, axis=2, keepdims=True) * scale + rb3
    m = jnp.max(attn, axis=1, keepdims=True)
    e = jnp.exp(attn - m)
    aw = e / jnp.sum(e, axis=1, keepdims=True)
    outh = jnp.sum(sv3 * aw, axis=1)
    acc = jnp.zeros((_TN, DIM), jnp.float32)
    for h in range(H):
        acc = acc + _dot(outh[h * _TN:(h + 1) * _TN, :],
                         wo[h * D:(h + 1) * D, :])
    out = acc + xt
    out = _ln(out, n1g_ref[...], n1b_ref[...])
    ffn = _gelu(_dot(out, fw1_ref[...]) + fb1_ref[...])
    ffn = _dot(ffn, fw2_ref[...]) + fb2_ref[...]
    out = _ln(out + ffn, n2g_ref[...], n2b_ref[...])
    out_ref[0] = out


def _row(a):
    return a.reshape(1, -1)


def kernel(x, xyz, params):
    B, C, N = x.shape
    H, P, D = NUM_HEADS, NUM_POINTS, HEAD_DIM
    p = params
    xt = jnp.transpose(x, (0, 2, 1))
    xyzt = jnp.transpose(xyz, (0, 2, 1))

    full = lambda a: pl.BlockSpec(a.shape, lambda *_: (0,) * a.ndim)

    ow1a = p['ow1'][:DIM]
    ow1b = p['ow1'][DIM:]
    w1 = [ow1a, ow1b, _row(p['ob1']), _row(p['oln_g']), _row(p['oln_b']),
          p['ow2'], _row(p['ob2']), p['ow3'], _row(p['ob3']),
          p['offset_scale'].reshape(1, 1),
          p['wq'], p['wk'], p['wv']]
    off, q, k, v = pl.pallas_call(
        _pre_kernel,
        grid=(B,),
        in_specs=[pl.BlockSpec((1, N, DIM), lambda b: (b, 0, 0)),
                  pl.BlockSpec((1, N, 3), lambda b: (b, 0, 0))]
                 + [full(a) for a in w1],
        out_shape=[jax.ShapeDtypeStruct((B, N, H * P * 3), jnp.float32),
                   jax.ShapeDtypeStruct((B, N, DIM), jnp.float32),
                   jax.ShapeDtypeStruct((B, N, DIM), jnp.float32),
                   jax.ShapeDtypeStruct((B, N, DIM), jnp.float32)],
        out_specs=[pl.BlockSpec((1, N, H * P * 3), lambda b: (b, 0, 0)),
                   pl.BlockSpec((1, N, DIM), lambda b: (b, 0, 0)),
                   pl.BlockSpec((1, N, DIM), lambda b: (b, 0, 0)),
                   pl.BlockSpec((1, N, DIM), lambda b: (b, 0, 0))],
    )(xt, xyz, *w1)

    off_heads = off.reshape(B, N, H, P * 3).transpose(0, 2, 1, 3) \
                   .reshape(B * H, N * P, 3)
    kvf = jnp.concatenate([k.reshape(B, N, H, D), v.reshape(B, N, H, D)],
                          axis=-1).transpose(0, 2, 1, 3).reshape(B * H, N, 2 * D)
    w2 = [p['rp_w1'], _row(p['rp_b1']), _row(p['rp_ln_g']), _row(p['rp_ln_b']),
          _row(p['rp_w2']), p['rp_b2'].reshape(1, 1)]
    nq = _QPTS * P
    skv, rb = pl.pallas_call(
        _knn_kernel,
        grid=(B * H, N // _QPTS),
        in_specs=[
            pl.BlockSpec((1, _QPTS, 3), lambda bh, c: (bh // H, c, 0)),
            pl.BlockSpec((1, nq, 3), lambda bh, c: (bh, c, 0)),
            pl.BlockSpec((1, 3, N), lambda bh, c: (bh // H, 0, 0)),
            pl.BlockSpec((1, N, 2 * D), lambda bh, c: (bh, 0, 0)),
        ] + [full(a) for a in w2],
        out_shape=[jax.ShapeDtypeStruct((B * H, N * P, 2 * D), jnp.float32),
                   jax.ShapeDtypeStruct((B * H, N * P, 1), jnp.float32)],
        out_specs=[pl.BlockSpec((1, nq, 2 * D), lambda bh, c: (bh, c, 0)),
                   pl.BlockSpec((1, nq, 1), lambda bh, c: (bh, c, 0))],
    )(xyz, off_heads, xyzt, kvf, *w2)

    q4 = q.reshape(B, N, H, D).transpose(0, 2, 1, 3)
    skv4 = skv.reshape(B, H, N * P, 2 * D)
    rb4 = rb.reshape(B, H, N * P, 1)
    w3 = [p['wo'], _row(p['n1_g']), _row(p['n1_b']),
          p['fw1'], _row(p['fb1']), p['fw2'], _row(p['fb2']),
          _row(p['n2_g']), _row(p['n2_b'])]
    out = pl.pallas_call(
        _attn_kernel,
        grid=(B, N // _TN),
        in_specs=[pl.BlockSpec((1, _TN, DIM), lambda b, n: (b, n, 0)),
                  pl.BlockSpec((1, H, _TN, D), lambda b, n: (b, 0, n, 0)),
                  pl.BlockSpec((1, H, _TN * P, 2 * D), lambda b, n: (b, 0, n, 0)),
                  pl.BlockSpec((1, H, _TN * P, 1), lambda b, n: (b, 0, n, 0))]
                 + [full(a) for a in w3],
        out_shape=jax.ShapeDtypeStruct((B, N, DIM), jnp.float32),
        out_specs=pl.BlockSpec((1, _TN, DIM), lambda b, n: (b, n, 0)),
    )(xt, q4, skv4, rb4, *w3)

    return jnp.transpose(out, (0, 2, 1))


# speedup vs baseline: 1.0340x; 1.0340x over previous
import jax
import jax.numpy as jnp
from jax.experimental import pallas as pl

DIM = 256
NUM_HEADS = 8
NUM_POINTS = 16
HEAD_DIM = DIM // NUM_HEADS

_QPTS = 64
_TN = 128


def _gelu(x):
    return 0.5 * x * (1.0 + jax.lax.erf(x * 0.7071067811865476))


def _ln(x, g, b):
    m = jnp.mean(x, axis=-1, keepdims=True)
    xc = x - m
    v = jnp.mean(xc * xc, axis=-1, keepdims=True)
    return xc * jax.lax.rsqrt(v + 1e-5) * g + b


def _dot(a, b):
    return jnp.dot(a, b, preferred_element_type=jnp.float32)


def _dot3(a, w):
    return (a[:, 0:1] * w[0:1, :] + a[:, 1:2] * w[1:2, :]
            + a[:, 2:3] * w[2:3, :])


def _pre_kernel(xt_ref, xyz_ref, ow1a_ref, ow1b_ref, ob1_ref, olng_ref,
                olnb_ref, ow2_ref, ob2_ref, ow3_ref, ob3_ref, scale_ref,
                wq_ref, wk_ref, wv_ref,
                off_ref, q_ref, k_ref, v_ref):
    xt = xt_ref[0]
    xyz = xyz_ref[0]
    h = _dot(xt, ow1a_ref[...]) + _dot3(xyz, ow1b_ref[...]) + ob1_ref[...]
    h = _gelu(_ln(h, olng_ref[...], olnb_ref[...]))
    h = _gelu(_dot(h, ow2_ref[...]) + ob2_ref[...])
    off = _dot(h, ow3_ref[...]) + ob3_ref[...]
    off = jnp.tanh(off) * jnp.abs(scale_ref[0, 0])
    off_ref[0] = off
    q_ref[0] = _dot(xt, wq_ref[...])
    k_ref[0] = _dot(xt, wk_ref[...])
    v_ref[0] = _dot(xt, wv_ref[...])


def _knn_kernel(qxyz_ref, off_ref, xyzt_ref, kvf_ref,
                rpw1_ref, rpb1_ref, rplng_ref, rplnb_ref, rpw2_ref, rpb2_ref,
                out_ref, rb_ref):
    off = off_ref[0]
    qxyz = jnp.broadcast_to(qxyz_ref[0][:, None, :],
                            (_QPTS, NUM_POINTS, 3)).reshape(_QPTS * NUM_POINTS, 3)
    qp = qxyz + off
    kx = xyzt_ref[0, 0:1, :]
    ky = xyzt_ref[0, 1:2, :]
    kz = xyzt_ref[0, 2:3, :]
    k3 = xyzt_ref[0]
    qhi = qp.astype(jnp.bfloat16).astype(jnp.float32)
    qlo = qp - qhi
    khi = k3.astype(jnp.bfloat16).astype(jnp.float32)
    klo = k3 - khi
    qaug = jnp.concatenate([-2.0 * qhi, -2.0 * qhi, -2.0 * qlo], axis=1)
    kaug = jnp.concatenate([khi, klo, khi], axis=0)
    qn = jnp.sum(qp * qp, axis=1, keepdims=True)
    kn = jnp.sum(k3 * k3, axis=0, keepdims=True)
    d2 = jnp.maximum(_dot(qaug, kaug) + qn + kn, 0.0)
    big = jnp.float32(3.0e38)
    nk = d2.shape[1]
    lw = nk // 8
    r1 = d2[:, 0:lw]
    r2 = jnp.full_like(r1, big)
    r3 = jnp.full_like(r1, big)
    for s_ in range(1, 8):
        xsl = d2[:, s_ * lw:(s_ + 1) * lw]
        t1 = jnp.minimum(r1, xsl)
        hi = jnp.maximum(r1, xsl)
        t2 = jnp.minimum(r2, hi)
        hi2 = jnp.maximum(r2, hi)
        t3 = jnp.minimum(r3, hi2)
        r1, r2, r3 = t1, t2, t3
    cand = jnp.concatenate([r1, r2, r3], axis=1)
    d1 = jnp.min(cand, axis=1, keepdims=True)
    m2 = jnp.min(jnp.where(cand > d1, cand, big), axis=1, keepdims=True)
    m3 = jnp.min(jnp.where(cand > m2, cand, big), axis=1, keepdims=True)
    dr0 = 1.0 / (d1 + 1e-8)
    dr1 = 1.0 / (m2 + 1e-8)
    dr2 = 1.0 / (m3 + 1e-8)
    rs = 1.0 / (dr0 + dr1 + dr2)
    w = jnp.where(d2 <= m3, (1.0 / (d2 + 1e-8)) * rs, 0.0)
    out_ref[0] = _dot(w, kvf_ref[0])

    rb = _gelu(_ln(_dot3(-off, rpw1_ref[...]) + rpb1_ref[...],
                   rplng_ref[...], rplnb_ref[...]))
    rb_ref[0] = (jnp.sum(rb * rpw2_ref[...], axis=1, keepdims=True)
                 + rpb2_ref[0, 0])


def _attn_kernel(xt_ref, q_ref, skv_ref, rb_ref,
                 wo_ref, n1g_ref, n1b_ref, fw1_ref, fb1_ref, fw2_ref, fb2_ref,
                 n2g_ref, n2b_ref, out_ref):
    H, P, D = NUM_HEADS, NUM_POINTS, HEAD_DIM
    xt = xt_ref[0]
    q4 = q_ref[0]
    skv4 = skv_ref[0]
    rb4 = rb_ref[0]
    scale = D ** -0.5
    wo = wo_ref[...]
    s3 = skv4.reshape(H * _TN, P, 2 * D)
    sk3 = s3[:, :, :D]
    sv3 = s3[:, :, D:]
    q3 = q4.reshape(H * _TN, 1, D)
    rb3 = rb4.reshape(H * _TN, P, 1)
    attn = jnp.sum(q3 * sk3, axis=2, keepdims=True) * scale + rb3
    m = jnp.max(attn, axis=1, keepdims=True)
    e = jnp.exp(attn - m)
    aw = e / jnp.sum(e, axis=1, keepdims=True)
    outh = jnp.sum(sv3 * aw, axis=1)
    acc = jnp.zeros((_TN, DIM), jnp.float32)
    for h in range(H):
        acc = acc + _dot(outh[h * _TN:(h + 1) * _TN, :],
                         wo[h * D:(h + 1) * D, :])
    out = acc + xt
    out = _ln(out, n1g_ref[...], n1b_ref[...])
    ffn = _gelu(_dot(out, fw1_ref[...]) + fb1_ref[...])
    ffn = _dot(ffn, fw2_ref[...]) + fb2_ref[...]
    out = _ln(out + ffn, n2g_ref[...], n2b_ref[...])
    out_ref[0] = out


def _row(a):
    return a.reshape(1, -1)


def kernel(x, xyz, params):
    B, C, N = x.shape
    H, P, D = NUM_HEADS, NUM_POINTS, HEAD_DIM
    p = params
    xt = jnp.transpose(x, (0, 2, 1))
    xyzt = jnp.transpose(xyz, (0, 2, 1))

    full = lambda a: pl.BlockSpec(a.shape, lambda *_: (0,) * a.ndim)

    ow1a = p['ow1'][:DIM]
    ow1b = p['ow1'][DIM:]
    w1 = [ow1a, ow1b, _row(p['ob1']), _row(p['oln_g']), _row(p['oln_b']),
          p['ow2'], _row(p['ob2']), p['ow3'], _row(p['ob3']),
          p['offset_scale'].reshape(1, 1),
          p['wq'], p['wk'], p['wv']]
    off, q, k, v = pl.pallas_call(
        _pre_kernel,
        grid=(B,),
        in_specs=[pl.BlockSpec((1, N, DIM), lambda b: (b, 0, 0)),
                  pl.BlockSpec((1, N, 3), lambda b: (b, 0, 0))]
                 + [full(a) for a in w1],
        out_shape=[jax.ShapeDtypeStruct((B, N, H * P * 3), jnp.float32),
                   jax.ShapeDtypeStruct((B, N, DIM), jnp.float32),
                   jax.ShapeDtypeStruct((B, N, DIM), jnp.float32),
                   jax.ShapeDtypeStruct((B, N, DIM), jnp.float32)],
        out_specs=[pl.BlockSpec((1, N, H * P * 3), lambda b: (b, 0, 0)),
                   pl.BlockSpec((1, N, DIM), lambda b: (b, 0, 0)),
                   pl.BlockSpec((1, N, DIM), lambda b: (b, 0, 0)),
                   pl.BlockSpec((1, N, DIM), lambda b: (b, 0, 0))],
    )(xt, xyz, *w1)

    off_heads = off.reshape(B, N, H, P * 3).transpose(0, 2, 1, 3) \
                   .reshape(B * H, N * P, 3)
    kvf = jnp.concatenate([k.reshape(B, N, H, D), v.reshape(B, N, H, D)],
                          axis=-1).transpose(0, 2, 1, 3).reshape(B * H, N, 2 * D)
    w2 = [p['rp_w1'], _row(p['rp_b1']), _row(p['rp_ln_g']), _row(p['rp_ln_b']),
          _row(p['rp_w2']), p['rp_b2'].reshape(1, 1)]
    nq = _QPTS * P
    skv, rb = pl.pallas_call(
        _knn_kernel,
        grid=(B * H, N // _QPTS),
        in_specs=[
            pl.BlockSpec((1, _QPTS, 3), lambda bh, c: (bh // H, c, 0)),
            pl.BlockSpec((1, nq, 3), lambda bh, c: (bh, c, 0)),
            pl.BlockSpec((1, 3, N), lambda bh, c: (bh // H, 0, 0)),
            pl.BlockSpec((1, N, 2 * D), lambda bh, c: (bh, 0, 0)),
        ] + [full(a) for a in w2],
        out_shape=[jax.ShapeDtypeStruct((B * H, N * P, 2 * D), jnp.float32),
                   jax.ShapeDtypeStruct((B * H, N * P, 1), jnp.float32)],
        out_specs=[pl.BlockSpec((1, nq, 2 * D), lambda bh, c: (bh, c, 0)),
                   pl.BlockSpec((1, nq, 1), lambda bh, c: (bh, c, 0))],
    )(xyz, off_heads, xyzt, kvf, *w2)

    q4 = q.reshape(B, N, H, D).transpose(0, 2, 1, 3)
    skv4 = skv.reshape(B, H, N * P, 2 * D)
    rb4 = rb.reshape(B, H, N * P, 1)
    w3 = [p['wo'], _row(p['n1_g']), _row(p['n1_b']),
          p['fw1'], _row(p['fb1']), p['fw2'], _row(p['fb2']),
          _row(p['n2_g']), _row(p['n2_b'])]
    out = pl.pallas_call(
        _attn_kernel,
        grid=(B, N // _TN),
        in_specs=[pl.BlockSpec((1, _TN, DIM), lambda b, n: (b, n, 0)),
                  pl.BlockSpec((1, H, _TN, D), lambda b, n: (b, 0, n, 0)),
                  pl.BlockSpec((1, H, _TN * P, 2 * D), lambda b, n: (b, 0, n, 0)),
                  pl.BlockSpec((1, H, _TN * P, 1), lambda b, n: (b, 0, n, 0))]
                 + [full(a) for a in w3],
        out_shape=jax.ShapeDtypeStruct((B, N, DIM), jnp.float32),
        out_specs=pl.BlockSpec((1, _TN, DIM), lambda b, n: (b, n, 0)),
    )(xt, q4, skv4, rb4, *w3)

    return jnp.transpose(out, (0, 2, 1))
